# baseline (speedup 1.0000x reference)
import functools

import jax
import jax.numpy as jnp
from jax import lax
from jax.experimental import pallas as pl
from jax.experimental.pallas import tpu as pltpu
from jax.experimental.pallas import tpu_sc as plsc

_EXPERTS = 64
_TOPK = 2
_NW = 32
_LANES = 16


def _logits_body(w_ref, h_ref, out_ref):
    out_ref[...] = lax.dot_general(
        w_ref[...], h_ref[...],
        dimension_numbers=(((1,), (1,)), ((), ())),
        preferred_element_type=jnp.float32)


def _compute_logits_t(hidden_flat, w, tm):
    n, h = hidden_flat.shape
    e = w.shape[0]
    return pl.pallas_call(
        _logits_body,
        grid=(n // tm,),
        in_specs=[
            pl.BlockSpec((e, h), lambda i: (0, 0)),
            pl.BlockSpec((tm, h), lambda i: (i, 0)),
        ],
        out_specs=pl.BlockSpec((e, tm), lambda i: (0, i)),
        out_shape=jax.ShapeDtypeStruct((e, n), jnp.float32),
        compiler_params=pltpu.CompilerParams(
            dimension_semantics=("arbitrary",)),
    )(w, hidden_flat)


def _make_sc_router(n):
    chunk = n // _NW
    ngroups = chunk // _LANES
    mesh = plsc.VectorSubcoreMesh(core_axis_name="c", subcore_axis_name="s")

    @functools.partial(
        pl.kernel, mesh=mesh,
        out_type=[
            jax.ShapeDtypeStruct((_TOPK, n), jnp.float32),
            jax.ShapeDtypeStruct((_TOPK, n), jnp.int32),
            jax.ShapeDtypeStruct((_NW, _EXPERTS, _LANES), jnp.float32),
            jax.ShapeDtypeStruct((_NW, _EXPERTS, _LANES), jnp.float32),
        ],
        scratch_types=[
            pltpu.VMEM((_EXPERTS, chunk), jnp.float32),
            pltpu.VMEM((_TOPK, chunk), jnp.float32),
            pltpu.VMEM((_TOPK, chunk), jnp.int32),
            pltpu.VMEM((_EXPERTS, _LANES), jnp.float32),
            pltpu.VMEM((_EXPERTS, _LANES), jnp.float32),
        ],
    )
    def sc_router(logits_hbm, gates_hbm, idx_hbm, psum_hbm, ppos_hbm,
                  lvm, gvm, ivm, asum, apos):
        wid = lax.axis_index("s") * 2 + lax.axis_index("c")
        base = wid * chunk
        pltpu.sync_copy(logits_hbm.at[:, pl.ds(base, chunk)], lvm)

        zf = jnp.zeros((_LANES,), jnp.float32)
        for ex in range(_EXPERTS):
            asum[ex, :] = zf
            apos[ex, :] = zf

        def group_body(g, carry):
            off = g * _LANES
            sl = pl.ds(off, _LANES)

            m1 = lvm[0, sl]
            i1 = jnp.zeros((_LANES,), jnp.int32)
            m2 = jnp.full((_LANES,), -jnp.inf, jnp.float32)
            i2 = jnp.zeros((_LANES,), jnp.int32)
            for ex in range(1, _EXPERTS):
                v = lvm[ex, sl]
                ei = jnp.full((_LANES,), ex, jnp.int32)
                gt1 = v > m1
                gt2 = v > m2
                nm2 = jnp.where(gt1, m1, jnp.where(gt2, v, m2))
                ni2 = jnp.where(gt1, i1, jnp.where(gt2, ei, i2))
                m1 = jnp.where(gt1, v, m1)
                i1 = jnp.where(gt1, ei, i1)
                m2, i2 = nm2, ni2

            e2 = jnp.exp(m2 - m1)
            den = 1.0 + e2
            gvm[0, sl] = 1.0 / den
            gvm[1, sl] = e2 / den
            ivm[0, sl] = i1
            ivm[1, sl] = i2

            ssum = jnp.zeros((_LANES,), jnp.float32)
            for ex in range(_EXPERTS):
                ssum = ssum + jnp.exp(lvm[ex, sl] - m1)
            rs = 1.0 / ssum
            for ex in range(_EXPERTS):
                p = jnp.exp(lvm[ex, sl] - m1) * rs
                asum[ex, :] += p
                apos[ex, :] += jnp.where(p > 0.0, 1.0, 0.0)
            return carry

        lax.fori_loop(0, ngroups, group_body, 0)

        pltpu.sync_copy(gvm, gates_hbm.at[:, pl.ds(base, chunk)])
        pltpu.sync_copy(ivm, idx_hbm.at[:, pl.ds(base, chunk)])
        pltpu.sync_copy(asum, psum_hbm.at[wid])
        pltpu.sync_copy(apos, ppos_hbm.at[wid])

    return sc_router


def _loss_body(psum_ref, ppos_ref, loss_ref, *, n_tok):
    s_e = jnp.sum(psum_ref[...], axis=(0, 2))
    c_e = jnp.sum(ppos_ref[...], axis=(0, 2))
    scale = jnp.float32(_EXPERTS) / (jnp.float32(n_tok) * jnp.float32(n_tok))
    loss_ref[...] = (scale * jnp.sum(s_e * c_e, keepdims=True)).reshape(1, 1)


def _combine_loss(psum, ppos, n_tok):
    return pl.pallas_call(
        functools.partial(_loss_body, n_tok=n_tok),
        out_shape=jax.ShapeDtypeStruct((1, 1), jnp.float32),
    )(psum, ppos)


def kernel(hidden_states, W):
    b, s, h = hidden_states.shape
    n = b * s
    hf = hidden_states.reshape(n, h)
    logits_t = _compute_logits_t(hf, W, tm=4096)
    gates_t, idx_t, psum, ppos = _make_sc_router(n)(logits_t)
    loss = _combine_loss(psum, ppos, n)
    gates = gates_t.T.reshape(b, s, _TOPK)
    idx = idx_t.T.reshape(b, s, _TOPK)
    return (gates, idx, loss[0, 0])


# speedup vs baseline: 2.0896x; 2.0896x over previous
import jax
import jax.numpy as jnp
from jax import lax
from jax.experimental import pallas as pl
from jax.experimental.pallas import tpu as pltpu

_EXPERTS = 64
_TOPK = 2


def _router_body(w_ref, h_ref, gates_ref, idx_ref, loss_ref, acc_sum, acc_pos):
    pid = pl.program_id(0)
    nprog = pl.num_programs(0)

    @pl.when(pid == 0)
    def _init():
        acc_sum[...] = jnp.zeros_like(acc_sum)
        acc_pos[...] = jnp.zeros_like(acc_pos)

    logits = lax.dot_general(
        w_ref[...], h_ref[...],
        dimension_numbers=(((1,), (1,)), ((), ())),
        preferred_element_type=jnp.float32)
    e, tm = logits.shape
    row = lax.broadcasted_iota(jnp.int32, (e, tm), 0)

    m1 = jnp.max(logits, axis=0, keepdims=True)
    i1 = jnp.min(jnp.where(logits == m1, row, e), axis=0, keepdims=True)
    masked = jnp.where(row == i1, jnp.float32(-jnp.inf), logits)
    m2 = jnp.max(masked, axis=0, keepdims=True)
    i2 = jnp.min(jnp.where(masked == m2, row, e), axis=0, keepdims=True)

    e2 = jnp.exp(m2 - m1)
    denom = 1.0 + e2
    gates_ref[...] = jnp.concatenate([1.0 / denom, e2 / denom], axis=0).T
    idx_ref[...] = jnp.concatenate([i1, i2], axis=0).T

    p = jnp.exp(logits - m1)
    pn = p / jnp.sum(p, axis=0, keepdims=True)
    acc_sum[...] += pn
    acc_pos[...] += (pn > 0).astype(jnp.float32)

    @pl.when(pid == nprog - 1)
    def _fin():
        n_tok = jnp.float32(nprog * tm)
        s_e = jnp.sum(acc_sum[...], axis=1)
        c_e = jnp.sum(acc_pos[...], axis=1)
        loss = (jnp.float32(e) / (n_tok * n_tok)) * jnp.sum(
            s_e * c_e, keepdims=True)
        loss_ref[...] = loss.reshape(1, 1)


def _run(hidden_flat, w, tm, interpret=False):
    n, h = hidden_flat.shape
    e = w.shape[0]
    grid = (n // tm,)
    return pl.pallas_call(
        _router_body,
        grid=grid,
        in_specs=[
            pl.BlockSpec((e, h), lambda i: (0, 0)),
            pl.BlockSpec((tm, h), lambda i: (i, 0)),
        ],
        out_specs=[
            pl.BlockSpec((tm, _TOPK), lambda i: (i, 0)),
            pl.BlockSpec((tm, _TOPK), lambda i: (i, 0)),
            pl.BlockSpec((1, 1), lambda i: (0, 0)),
        ],
        out_shape=[
            jax.ShapeDtypeStruct((n, _TOPK), jnp.float32),
            jax.ShapeDtypeStruct((n, _TOPK), jnp.int32),
            jax.ShapeDtypeStruct((1, 1), jnp.float32),
        ],
        scratch_shapes=[
            pltpu.VMEM((e, tm), jnp.float32),
            pltpu.VMEM((e, tm), jnp.float32),
        ],
        compiler_params=pltpu.CompilerParams(
            dimension_semantics=("arbitrary",)),
        interpret=interpret,
    )(w, hidden_flat)


def kernel(hidden_states, W):
    b, s, h = hidden_states.shape
    hf = hidden_states.reshape(b * s, h)
    gates_o, idx_o, loss = _run(hf, W, tm=4096)
    gates = gates_o.reshape(b, s, _TOPK)
    idx = idx_o.reshape(b, s, _TOPK)
    return (gates, idx, loss[0, 0])


# speedup vs baseline: 3.6039x; 1.7247x over previous
import jax
import jax.numpy as jnp
from jax import lax
from jax.experimental import pallas as pl
from jax.experimental.pallas import tpu as pltpu

_EXPERTS = 64
_TOPK = 2


def _router_body(w_ref, ha_ref, hb_ref, gates_ref, idx_ref, loss_ref,
                 acc_sum, acc_pos):
    pid = pl.program_id(0)
    nprog = pl.num_programs(0)

    @pl.when(pid == 0)
    def _init():
        acc_sum[...] = jnp.zeros_like(acc_sum)
        acc_pos[...] = jnp.zeros_like(acc_pos)

    dn = (((1,), (1,)), ((), ()))
    hh = ha_ref.shape[1]
    logits = lax.dot_general(
        w_ref[:, :hh], ha_ref[...], dimension_numbers=dn,
        preferred_element_type=jnp.float32) + lax.dot_general(
        w_ref[:, hh:], hb_ref[...], dimension_numbers=dn,
        preferred_element_type=jnp.float32)
    e, tm = logits.shape
    row = lax.broadcasted_iota(jnp.int32, (e, tm), 0)

    m1 = jnp.max(logits, axis=0, keepdims=True)
    i1 = jnp.min(jnp.where(logits == m1, row, e), axis=0, keepdims=True)
    masked = jnp.where(row == i1, jnp.float32(-jnp.inf), logits)
    m2 = jnp.max(masked, axis=0, keepdims=True)
    i2 = jnp.min(jnp.where(masked == m2, row, e), axis=0, keepdims=True)

    e2 = jnp.exp(m2 - m1)
    denom = 1.0 + e2
    gates_ref[...] = jnp.concatenate([1.0 / denom, e2 / denom], axis=0)
    idx_ref[...] = jnp.concatenate([i1, i2], axis=0)

    p = jnp.exp(logits - m1)
    pn = p / jnp.sum(p, axis=0, keepdims=True)
    acc_sum[...] += pn
    acc_pos[...] += (pn > 0).astype(jnp.float32)

    @pl.when(pid == nprog - 1)
    def _fin():
        n_tok = jnp.float32(nprog * tm)
        s_e = jnp.sum(acc_sum[...], axis=1)
        c_e = jnp.sum(acc_pos[...], axis=1)
        loss = (jnp.float32(e) / (n_tok * n_tok)) * jnp.sum(
            s_e * c_e, keepdims=True)
        loss_ref[...] = loss.reshape(1, 1)


def _run(hidden_flat, w, tm, interpret=False):
    n, h = hidden_flat.shape
    e = w.shape[0]
    grid = (n // tm,)
    return pl.pallas_call(
        _router_body,
        grid=grid,
        in_specs=[
            pl.BlockSpec((e, h), lambda i: (0, 0)),
            pl.BlockSpec((tm, h // 2), lambda i: (i, 0)),
            pl.BlockSpec((tm, h // 2), lambda i: (i, 1)),
        ],
        out_specs=[
            pl.BlockSpec((_TOPK, tm), lambda i: (0, i)),
            pl.BlockSpec((_TOPK, tm), lambda i: (0, i)),
            pl.BlockSpec((1, 1), lambda i: (0, 0)),
        ],
        out_shape=[
            jax.ShapeDtypeStruct((_TOPK, n), jnp.float32),
            jax.ShapeDtypeStruct((_TOPK, n), jnp.int32),
            jax.ShapeDtypeStruct((1, 1), jnp.float32),
        ],
        scratch_shapes=[
            pltpu.VMEM((e, tm), jnp.float32),
            pltpu.VMEM((e, tm), jnp.float32),
        ],
        compiler_params=pltpu.CompilerParams(
            dimension_semantics=("arbitrary",)),
        interpret=interpret,
    )(w, hidden_flat, hidden_flat)


def kernel(hidden_states, W):
    b, s, h = hidden_states.shape
    hf = hidden_states.reshape(b * s, h)
    gates_t, idx_t, loss = _run(hf, W, tm=4096)
    gates = gates_t.T.reshape(b, s, _TOPK)
    idx = idx_t.T.reshape(b, s, _TOPK)
    return (gates, idx, loss[0, 0])
